# Initial kernel scaffold; baseline (speedup 1.0000x reference)
import jax
import jax.numpy as jnp
from jax.experimental import pallas as pl


def kernel(x, edge_index, W_rel, b_rel, W_root):
    raise NotImplementedError("write your pallas kernel here")


# speedup vs baseline: 4.5204x; 4.5204x over previous
import functools

import jax
import jax.numpy as jnp
from jax import lax
from jax.experimental import pallas as pl
from jax.experimental.pallas import tpu as pltpu
from jax.experimental.pallas import tpu_sc as plsc

N_NODES = 10000
N_EDGES = 320000
D = 128

NUM_TILES = 32
CHUNK = 128
CHUNKS_PER_TILE = 80
STAGE = 40
E_PAD = NUM_TILES * CHUNKS_PER_TILE * CHUNK
N_PAD = 10112
ROWS_PER_TILE = N_PAD // 16


def _sc_aggregate(src2d, dst2d, x):
    mesh = plsc.VectorSubcoreMesh(core_axis_name="c", subcore_axis_name="s")

    @functools.partial(
        pl.kernel,
        mesh=mesh,
        out_type=jax.ShapeDtypeStruct((2, N_PAD, D), jnp.float32),
        scratch_types=[
            pltpu.VMEM((STAGE, CHUNK), jnp.int32),
            pltpu.VMEM((STAGE, CHUNK), jnp.int32),
            pltpu.VMEM((CHUNK, D), jnp.float32),
            pltpu.VMEM((CHUNK, D), jnp.float32),
            pltpu.VMEM_SHARED((N_PAD, D), jnp.float32),
            pltpu.SemaphoreType.DMA,
            pltpu.SemaphoreType.DMA,
        ],
    )
    def agg_kernel(src_hbm, dst_hbm, x_hbm, out_hbm,
                   src_v, dst_v, buf0, buf1, agg_sh, sem0, sem1):
        c = lax.axis_index("c")
        s = lax.axis_index("s")
        wid = s * 2 + c

        zbuf = buf0

        def zero_body(i, carry):
            zbuf[i // 8, pl.ds((i % 8) * 16, 16)] = jnp.zeros((16,), jnp.float32)
            return carry
        lax.fori_loop(0, CHUNK * D // 16, zero_body, 0)
        zbase = s * ROWS_PER_TILE
        nfull = ROWS_PER_TILE // CHUNK
        for k in range(nfull):
            pltpu.sync_copy(zbuf, agg_sh.at[pl.ds(zbase + k * CHUNK, CHUNK)])
        rem = ROWS_PER_TILE - nfull * CHUNK
        pltpu.sync_copy(zbuf.at[pl.ds(0, rem)],
                        agg_sh.at[pl.ds(zbase + nfull * CHUNK, rem)])
        plsc.subcore_barrier()

        bufs = (buf0, buf1)
        sems = (sem0, sem1)
        cbase = wid * CHUNKS_PER_TILE

        for stage in range(CHUNKS_PER_TILE // STAGE):
            pltpu.sync_copy(src_hbm.at[pl.ds(cbase + stage * STAGE, STAGE)], src_v)
            pltpu.sync_copy(dst_hbm.at[pl.ds(cbase + stage * STAGE, STAGE)], dst_v)

            pltpu.async_copy(x_hbm.at[src_v.at[0]], bufs[0], sems[0])

            def chunk_body(jj, carry):
                for b in range(2):
                    j = jj * 2 + b
                    nxt = j + 1

                    @pl.when(nxt < STAGE)
                    def _():
                        pltpu.async_copy(x_hbm.at[src_v.at[nxt]],
                                         bufs[1 - b], sems[1 - b])

                    pltpu.make_async_copy(x_hbm.at[src_v.at[j]],
                                          bufs[b], sems[b]).wait()
                    pltpu.sync_copy(bufs[b], agg_sh.at[dst_v.at[j]], add=True)
                return carry

            lax.fori_loop(0, STAGE // 2, chunk_body, 0)
        plsc.subcore_barrier()

        pltpu.sync_copy(agg_sh.at[pl.ds(zbase, ROWS_PER_TILE)],
                        out_hbm.at[c, pl.ds(zbase, ROWS_PER_TILE)])

    return agg_kernel(src2d, dst2d, x)


def _tc_combine(agg2, x, W_rel, b_rel2, W_root):
    blk = 1000
    grid = N_NODES // blk

    def body(a_ref, x_ref, wrel_ref, wroot_ref, b_ref, o_ref):
        agg = a_ref[0] + a_ref[1]
        dn = (((1,), (1,)), ((), ()))
        o_ref[...] = (
            lax.dot_general(agg, wrel_ref[...], dn,
                            preferred_element_type=jnp.float32)
            + lax.dot_general(x_ref[...], wroot_ref[...], dn,
                              preferred_element_type=jnp.float32)
            + b_ref[...]
        )

    return pl.pallas_call(
        body,
        grid=(grid,),
        in_specs=[
            pl.BlockSpec((2, blk, D), lambda i: (0, i, 0)),
            pl.BlockSpec((blk, D), lambda i: (i, 0)),
            pl.BlockSpec((D, D), lambda i: (0, 0)),
            pl.BlockSpec((D, D), lambda i: (0, 0)),
            pl.BlockSpec((1, D), lambda i: (0, 0)),
        ],
        out_specs=pl.BlockSpec((blk, D), lambda i: (i, 0)),
        out_shape=jax.ShapeDtypeStruct((N_NODES, D), jnp.float32),
    )(agg2, x, W_rel, W_root, b_rel2)


def kernel(x, edge_index, W_rel, b_rel, W_root):
    src = edge_index[0].astype(jnp.int32)
    dst = edge_index[1].astype(jnp.int32)
    pad = E_PAD - N_EDGES
    src2d = jnp.concatenate(
        [src, jnp.zeros((pad,), jnp.int32)]).reshape(-1, CHUNK)
    dst2d = jnp.concatenate(
        [dst, jnp.full((pad,), N_NODES, jnp.int32)]).reshape(-1, CHUNK)
    agg2 = _sc_aggregate(src2d, dst2d, x)
    return _tc_combine(agg2, x, W_rel, b_rel.reshape(1, D), W_root)


# speedup vs baseline: 4.6123x; 1.0203x over previous
import functools

import jax
import jax.numpy as jnp
from jax import lax
from jax.experimental import pallas as pl
from jax.experimental.pallas import tpu as pltpu
from jax.experimental.pallas import tpu_sc as plsc

N_NODES = 10000
N_EDGES = 320000
D = 128

NUM_TILES = 32
CHUNK = 128
STAGE = 40
FAST_CORE = 0
FAST_CHUNKS = 120
SLOW_CHUNKS = 40
TOTAL_CHUNKS = 16 * (FAST_CHUNKS + SLOW_CHUNKS)
E_PAD = TOTAL_CHUNKS * CHUNK
N_PAD = 10112
ROWS_PER_TILE = N_PAD // 16


def _sc_aggregate(src2d, dst2d, x):
    mesh = plsc.VectorSubcoreMesh(core_axis_name="c", subcore_axis_name="s")

    @functools.partial(
        pl.kernel,
        mesh=mesh,
        out_type=jax.ShapeDtypeStruct((2, N_PAD, D), jnp.float32),
        scratch_types=[
            pltpu.VMEM((STAGE, CHUNK), jnp.int32),
            pltpu.VMEM((STAGE, CHUNK), jnp.int32),
            pltpu.VMEM((CHUNK, D), jnp.float32),
            pltpu.VMEM((CHUNK, D), jnp.float32),
            pltpu.VMEM_SHARED((N_PAD, D), jnp.float32),
            pltpu.SemaphoreType.DMA,
            pltpu.SemaphoreType.DMA,
        ],
    )
    def agg_kernel(src_hbm, dst_hbm, x_hbm, out_hbm,
                   src_v, dst_v, buf0, buf1, agg_sh, sem0, sem1):
        c = lax.axis_index("c")
        s = lax.axis_index("s")
        wid = s * 2 + c

        zbuf = buf0

        def zero_body(i, carry):
            zbuf[i // 8, pl.ds((i % 8) * 16, 16)] = jnp.zeros((16,), jnp.float32)
            return carry
        lax.fori_loop(0, CHUNK * D // 16, zero_body, 0)
        zbase = s * ROWS_PER_TILE
        nfull = ROWS_PER_TILE // CHUNK
        for k in range(nfull):
            pltpu.sync_copy(zbuf, agg_sh.at[pl.ds(zbase + k * CHUNK, CHUNK)])
        rem = ROWS_PER_TILE - nfull * CHUNK
        pltpu.sync_copy(zbuf.at[pl.ds(0, rem)],
                        agg_sh.at[pl.ds(zbase + nfull * CHUNK, rem)])
        plsc.subcore_barrier()

        bufs = (buf0, buf1)
        sems = (sem0, sem1)
        is_fast = c == FAST_CORE
        cbase = jnp.where(is_fast, s * FAST_CHUNKS,
                          16 * FAST_CHUNKS + s * SLOW_CHUNKS)
        nstages = jnp.where(is_fast, FAST_CHUNKS // STAGE, SLOW_CHUNKS // STAGE)

        def stage_body(stage, carry):
            sb = cbase + stage * STAGE
            pltpu.sync_copy(src_hbm.at[pl.ds(sb, STAGE)], src_v)
            pltpu.sync_copy(dst_hbm.at[pl.ds(sb, STAGE)], dst_v)

            pltpu.async_copy(x_hbm.at[src_v.at[0]], bufs[0], sems[0])

            def chunk_body(jj, carry2):
                for b in range(2):
                    j = jj * 2 + b
                    nxt = j + 1

                    @pl.when(nxt < STAGE)
                    def _():
                        pltpu.async_copy(x_hbm.at[src_v.at[nxt]],
                                         bufs[1 - b], sems[1 - b])

                    pltpu.make_async_copy(x_hbm.at[src_v.at[j]],
                                          bufs[b], sems[b]).wait()
                    pltpu.sync_copy(bufs[b], agg_sh.at[dst_v.at[j]], add=True)
                return carry2

            lax.fori_loop(0, STAGE // 2, chunk_body, 0)
            return carry

        lax.fori_loop(0, nstages, stage_body, 0)
        plsc.subcore_barrier()

        pltpu.sync_copy(agg_sh.at[pl.ds(zbase, ROWS_PER_TILE)],
                        out_hbm.at[c, pl.ds(zbase, ROWS_PER_TILE)])

    return agg_kernel(src2d, dst2d, x)


def _tc_combine(agg2, x, W_rel, b_rel2, W_root):
    blk = 1000
    grid = N_NODES // blk

    def body(a_ref, x_ref, wrel_ref, wroot_ref, b_ref, o_ref):
        agg = a_ref[0] + a_ref[1]
        dn = (((1,), (1,)), ((), ()))
        o_ref[...] = (
            lax.dot_general(agg, wrel_ref[...], dn,
                            preferred_element_type=jnp.float32)
            + lax.dot_general(x_ref[...], wroot_ref[...], dn,
                              preferred_element_type=jnp.float32)
            + b_ref[...]
        )

    return pl.pallas_call(
        body,
        grid=(grid,),
        in_specs=[
            pl.BlockSpec((2, blk, D), lambda i: (0, i, 0)),
            pl.BlockSpec((blk, D), lambda i: (i, 0)),
            pl.BlockSpec((D, D), lambda i: (0, 0)),
            pl.BlockSpec((D, D), lambda i: (0, 0)),
            pl.BlockSpec((1, D), lambda i: (0, 0)),
        ],
        out_specs=pl.BlockSpec((blk, D), lambda i: (i, 0)),
        out_shape=jax.ShapeDtypeStruct((N_NODES, D), jnp.float32),
    )(agg2, x, W_rel, W_root, b_rel2)


def kernel(x, edge_index, W_rel, b_rel, W_root):
    src = edge_index[0].astype(jnp.int32)
    dst = edge_index[1].astype(jnp.int32)
    pad = E_PAD - N_EDGES
    src2d = jnp.concatenate(
        [src, jnp.zeros((pad,), jnp.int32)]).reshape(-1, CHUNK)
    dst2d = jnp.concatenate(
        [dst, jnp.full((pad,), N_NODES, jnp.int32)]).reshape(-1, CHUNK)
    agg2 = _sc_aggregate(src2d, dst2d, x)
    return _tc_combine(agg2, x, W_rel, b_rel.reshape(1, D), W_root)
